# baseline (speedup 1.0000x reference)
import functools

import jax
import jax.numpy as jnp
from jax import lax
from jax.experimental import pallas as pl
from jax.experimental.pallas import tpu as pltpu
from jax.experimental.pallas import tpu_sc as plsc

_NC, _NS, _LANES = 2, 16, 16
_NW = _NC * _NS


def kernel(tokens, pos_emb):
    B, L, D = tokens.shape
    rows_per_w = L // _NW
    C = 16
    n_q = rows_per_w // C
    items = [(q, b) for q in range(n_q) for b in range(B)]
    n_items = len(items)
    NBUF = 5
    LOOK = 3

    tok2d = tokens.reshape(B * L, D)

    mesh = plsc.VectorSubcoreMesh(core_axis_name="c", subcore_axis_name="s")

    @functools.partial(
        pl.kernel,
        out_type=jax.ShapeDtypeStruct((B * L, D), jnp.float32),
        mesh=mesh,
        scratch_types=[
            [pltpu.VMEM((C, D), jnp.float32) for _ in range(2)],
            [pltpu.VMEM((C, D), jnp.float32) for _ in range(NBUF)],
            [pltpu.SemaphoreType.DMA for _ in range(2)],
            [pltpu.SemaphoreType.DMA for _ in range(2 * NBUF)],
        ],
    )
    def sc_add(tok_hbm, pe_hbm, out_hbm, pe_bufs, bufs, pe_sems, sems):
        wid = lax.axis_index("c") * _NS + lax.axis_index("s")
        base = wid * rows_per_w

        def tok_row(k):
            q, b = items[k]
            return b * L + base + q * C

        def pe_fetch(q):
            return pltpu.async_copy(
                pe_hbm.at[pl.ds(base + q * C, C)], pe_bufs[q % 2], pe_sems[q % 2]
            )

        in_d = [None] * (n_items + NBUF)
        out_d = [None] * n_items
        pe_d = [None] * (n_q + 1)
        pe_d[0] = pe_fetch(0)
        for k in range(LOOK):
            in_d[k] = pltpu.async_copy(
                tok_hbm.at[pl.ds(tok_row(k), C)], bufs[k % NBUF], sems[k % NBUF]
            )
        for k in range(n_items):
            p = k % NBUF
            q, b = items[k]
            if b == 0:
                pe_d[q].wait()
                if q + 1 < n_q:
                    pe_d[q + 1] = pe_fetch(q + 1)
            if k + LOOK < n_items:
                if k + LOOK >= NBUF:
                    out_d[k + LOOK - NBUF].wait()
                in_d[k + LOOK] = pltpu.async_copy(
                    tok_hbm.at[pl.ds(tok_row(k + LOOK), C)],
                    bufs[(k + LOOK) % NBUF],
                    sems[(k + LOOK) % NBUF],
                )
            in_d[k].wait()
            pe_buf = pe_bufs[q % 2]

            @plsc.parallel_loop(0, C * D, step=_LANES, unroll=8)
            def _(i):
                r = i // D
                j = i % D
                plsc.addupdate(
                    bufs[p].at[r, pl.ds(j, _LANES)],
                    pe_buf[r, pl.ds(j, _LANES)],
                )

            out_d[k] = pltpu.async_copy(
                bufs[p], out_hbm.at[pl.ds(tok_row(k), C)], sems[NBUF + p]
            )
        for k in range(n_items - NBUF, n_items):
            out_d[k].wait()

    out = sc_add(tok2d, pos_emb)
    return out.reshape(B, L, D)


# speedup vs baseline: 1.0069x; 1.0069x over previous
import functools

import jax
import jax.numpy as jnp
from jax import lax
from jax.experimental import pallas as pl
from jax.experimental.pallas import tpu as pltpu
from jax.experimental.pallas import tpu_sc as plsc

_NC, _NS, _LANES = 2, 16, 16
_NW = _NC * _NS


def kernel(tokens, pos_emb):
    B, L, D = tokens.shape
    rows_per_w = L // _NW
    C = 16
    n_q = rows_per_w // C
    items = [(q, b) for q in range(n_q) for b in range(B)]
    n_items = len(items)
    NBUF = 5
    LOOK = 3

    tok2d = tokens.reshape(B * L, D)

    mesh = plsc.VectorSubcoreMesh(core_axis_name="c", subcore_axis_name="s")

    @functools.partial(
        pl.kernel,
        out_type=jax.ShapeDtypeStruct((B * L, D), jnp.float32),
        mesh=mesh,
        scratch_types=[
            [pltpu.VMEM((C, D), jnp.float32) for _ in range(2)],
            [pltpu.VMEM((C, D), jnp.float32) for _ in range(NBUF)],
            [pltpu.SemaphoreType.DMA for _ in range(2)],
            [pltpu.SemaphoreType.DMA for _ in range(2 * NBUF)],
        ],
    )
    def sc_add(tok_hbm, pe_hbm, out_hbm, pe_bufs, bufs, pe_sems, sems):
        wid = lax.axis_index("s") * _NC + lax.axis_index("c")
        base = wid * rows_per_w

        def tok_row(k):
            q, b = items[k]
            return b * L + base + q * C

        def pe_fetch(q):
            return pltpu.async_copy(
                pe_hbm.at[pl.ds(base + q * C, C)], pe_bufs[q % 2], pe_sems[q % 2]
            )

        in_d = [None] * (n_items + NBUF)
        out_d = [None] * n_items
        pe_d = [None] * (n_q + 1)
        pe_d[0] = pe_fetch(0)
        for k in range(LOOK):
            in_d[k] = pltpu.async_copy(
                tok_hbm.at[pl.ds(tok_row(k), C)], bufs[k % NBUF], sems[k % NBUF]
            )
        for k in range(n_items):
            p = k % NBUF
            q, b = items[k]
            if b == 0:
                pe_d[q].wait()
                if q + 1 < n_q:
                    pe_d[q + 1] = pe_fetch(q + 1)
            if k + LOOK < n_items:
                if k + LOOK >= NBUF:
                    out_d[k + LOOK - NBUF].wait()
                in_d[k + LOOK] = pltpu.async_copy(
                    tok_hbm.at[pl.ds(tok_row(k + LOOK), C)],
                    bufs[(k + LOOK) % NBUF],
                    sems[(k + LOOK) % NBUF],
                )
            in_d[k].wait()
            pe_buf = pe_bufs[q % 2]

            @plsc.parallel_loop(0, C * D, step=_LANES, unroll=8)
            def _(i):
                r = i // D
                j = i % D
                plsc.addupdate(
                    bufs[p].at[r, pl.ds(j, _LANES)],
                    pe_buf[r, pl.ds(j, _LANES)],
                )

            out_d[k] = pltpu.async_copy(
                bufs[p], out_hbm.at[pl.ds(tok_row(k), C)], sems[NBUF + p]
            )
        for k in range(n_items - NBUF, n_items):
            out_d[k].wait()

    out = sc_add(tok2d, pos_emb)
    return out.reshape(B, L, D)


# speedup vs baseline: 1.0196x; 1.0127x over previous
import functools

import jax
import jax.numpy as jnp
from jax import lax
from jax.experimental import pallas as pl
from jax.experimental.pallas import tpu as pltpu
from jax.experimental.pallas import tpu_sc as plsc

_NC, _NS, _LANES = 2, 16, 16
_NW = _NC * _NS


def kernel(tokens, pos_emb):
    B, L, D = tokens.shape
    rows_per_w = L // _NW
    C = 16
    n_q = rows_per_w // C
    items = [(q, b) for q in range(n_q) for b in range(B)]
    n_items = len(items)
    NBUF = 5
    LOOK = 3

    tok2d = tokens.reshape(B * L, D)

    mesh = plsc.VectorSubcoreMesh(core_axis_name="c", subcore_axis_name="s")

    @functools.partial(
        pl.kernel,
        out_type=jax.ShapeDtypeStruct((B * L, D), jnp.float32),
        mesh=mesh,
        scratch_types=[
            [pltpu.VMEM((C, D), jnp.float32) for _ in range(2)],
            [pltpu.VMEM((C, D), jnp.float32) for _ in range(NBUF)],
            [pltpu.SemaphoreType.DMA for _ in range(2)],
            [pltpu.SemaphoreType.DMA for _ in range(2 * NBUF)],
        ],
    )
    def sc_add(tok_hbm, pe_hbm, out_hbm, pe_bufs, bufs, pe_sems, sems):
        wid = lax.axis_index("s") * _NC + lax.axis_index("c")
        base = wid * rows_per_w

        def tok_row(k):
            q, b = items[k]
            return b * L + base + q * C

        def pe_fetch(q):
            return pltpu.async_copy(
                pe_hbm.at[pl.ds(base + q * C, C)], pe_bufs[q % 2], pe_sems[q % 2]
            )

        in_d = [None] * (n_items + NBUF)
        out_d = [None] * n_items
        pe_d = [None] * (n_q + 1)
        pe_d[0] = pe_fetch(0)
        for k in range(LOOK):
            in_d[k] = pltpu.async_copy(
                tok_hbm.at[pl.ds(tok_row(k), C)], bufs[k % NBUF], sems[k % NBUF]
            )
        for k in range(n_items):
            p = k % NBUF
            q, b = items[k]
            if b == 0:
                pe_d[q].wait()
                if q + 1 < n_q:
                    pe_d[q + 1] = pe_fetch(q + 1)
            if k + LOOK < n_items:
                if k + LOOK >= NBUF:
                    out_d[k + LOOK - NBUF].wait()
                in_d[k + LOOK] = pltpu.async_copy(
                    tok_hbm.at[pl.ds(tok_row(k + LOOK), C)],
                    bufs[(k + LOOK) % NBUF],
                    sems[(k + LOOK) % NBUF],
                )
            in_d[k].wait()
            pe_buf = pe_bufs[q % 2]

            @plsc.parallel_loop(0, C * D, step=_LANES, unroll=4)
            def _(i):
                r = i // D
                j = i % D
                plsc.addupdate(
                    bufs[p].at[r, pl.ds(j, _LANES)],
                    pe_buf[r, pl.ds(j, _LANES)],
                )

            out_d[k] = pltpu.async_copy(
                bufs[p], out_hbm.at[pl.ds(tok_row(k), C)], sems[NBUF + p]
            )
        for k in range(n_items - NBUF, n_items):
            out_d[k].wait()

    out = sc_add(tok2d, pos_emb)
    return out.reshape(B, L, D)
